# baseline (speedup 1.0000x reference)
import functools

import numpy as np
import jax
import jax.numpy as jnp
from jax.experimental import pallas as pl
from jax.experimental.pallas import tpu as pltpu

_N_BINS = 15


def _ece_body(logits_ref, labels_ref, out_ref, cnt_ref, sconf_ref, sacc_ref,
              *, n_steps, n_rows_total):
    i = pl.program_id(0)

    @pl.when(i == 0)
    def _init():
        cnt_ref[...] = jnp.zeros_like(cnt_ref)
        sconf_ref[...] = jnp.zeros_like(sconf_ref)
        sacc_ref[...] = jnp.zeros_like(sacc_ref)

    x = logits_ref[...]
    y = labels_ref[...]
    r, l = x.shape

    mx = jnp.max(x, axis=1, keepdims=True)
    my = jnp.max(y, axis=1, keepdims=True)
    iota = jax.lax.broadcasted_iota(jnp.int32, (r, l), 1)
    big = jnp.int32(l)
    px = jnp.min(jnp.where(x >= mx, iota, big), axis=1, keepdims=True)
    py = jnp.min(jnp.where(y >= my, iota, big), axis=1, keepdims=True)

    conf = jax.nn.sigmoid(mx)
    acc = (px == py).astype(jnp.float32)

    lane = jax.lax.broadcasted_iota(jnp.int32, (1, 16), 1)
    lane_f = lane.astype(jnp.float32)
    lowers = jnp.where(lane < _N_BINS, lane_f / 15.0, 2.0)
    uppers = jnp.where(lane < _N_BINS, (lane_f + 1.0) / 15.0, -2.0)
    mask = ((conf > lowers) & (conf <= uppers)).astype(jnp.float32)
    cnt_ref[...] += mask
    sconf_ref[...] += mask * conf
    sacc_ref[...] += mask * acc

    @pl.when(i == n_steps - 1)
    def _finish():
        cnt = jnp.sum(cnt_ref[...], axis=0, keepdims=True)
        sc = jnp.sum(sconf_ref[...], axis=0, keepdims=True)
        sa = jnp.sum(sacc_ref[...], axis=0, keepdims=True)
        prop = cnt / jnp.float32(n_rows_total)
        safe = jnp.maximum(cnt, 1.0)
        contrib = jnp.abs(sc / safe - sa / safe) * prop
        contrib = jnp.where(cnt > 0.0, contrib, 0.0)
        out_ref[...] = jnp.sum(contrib, axis=1, keepdims=True)


@functools.partial(jax.jit, static_argnames=("block_rows",))
def _ece_pallas(logits, labels, block_rows):
    n_rows, n_cols = logits.shape
    n_steps = n_rows // block_rows
    body = functools.partial(_ece_body, n_steps=n_steps, n_rows_total=n_rows)
    out = pl.pallas_call(
        body,
        grid=(n_steps,),
        in_specs=[
            pl.BlockSpec((block_rows, n_cols), lambda i: (i, 0)),
            pl.BlockSpec((block_rows, n_cols), lambda i: (i, 0)),
        ],
        out_specs=pl.BlockSpec((1, 1), lambda i: (0, 0)),
        out_shape=jax.ShapeDtypeStruct((1, 1), jnp.float32),
        scratch_shapes=[
            pltpu.VMEM((block_rows, 16), jnp.float32),
            pltpu.VMEM((block_rows, 16), jnp.float32),
            pltpu.VMEM((block_rows, 16), jnp.float32),
        ],
        compiler_params=pltpu.CompilerParams(
            dimension_semantics=("arbitrary",),
        ),
    )(logits, labels)
    return out.reshape(1)


def kernel(logits, labels):
    n_rows = logits.shape[0]
    block_rows = 512 if n_rows % 512 == 0 else n_rows
    return _ece_pallas(logits, labels, block_rows)


# speedup vs baseline: 1.0437x; 1.0437x over previous
import functools

import numpy as np
import jax
import jax.numpy as jnp
from jax.experimental import pallas as pl
from jax.experimental.pallas import tpu as pltpu

_N_BINS = 15


def _ece_body(logits_ref, labels_ref, out_ref, cnt_ref, sconf_ref, sacc_ref,
              *, n_steps, n_rows_total):
    i = pl.program_id(0)

    @pl.when(i == 0)
    def _init():
        cnt_ref[...] = jnp.zeros_like(cnt_ref)
        sconf_ref[...] = jnp.zeros_like(sconf_ref)
        sacc_ref[...] = jnp.zeros_like(sacc_ref)

    x = logits_ref[...]
    y = labels_ref[...]
    r, l = x.shape

    mx = jnp.max(x, axis=1, keepdims=True)
    my = jnp.max(y, axis=1, keepdims=True)
    hit = (x >= mx) & (y >= my)
    acc = jnp.any(hit, axis=1, keepdims=True).astype(jnp.float32)
    conf = jax.nn.sigmoid(mx)

    lane = jax.lax.broadcasted_iota(jnp.int32, (1, 16), 1)
    lane_f = lane.astype(jnp.float32)
    lowers = jnp.where(lane < _N_BINS, lane_f / 15.0, 2.0)
    uppers = jnp.where(lane < _N_BINS, (lane_f + 1.0) / 15.0, -2.0)
    mask = ((conf > lowers) & (conf <= uppers)).astype(jnp.float32)
    cnt_ref[...] += mask
    sconf_ref[...] += mask * conf
    sacc_ref[...] += mask * acc

    @pl.when(i == n_steps - 1)
    def _finish():
        cnt = jnp.sum(cnt_ref[...], axis=0, keepdims=True)
        sc = jnp.sum(sconf_ref[...], axis=0, keepdims=True)
        sa = jnp.sum(sacc_ref[...], axis=0, keepdims=True)
        prop = cnt / jnp.float32(n_rows_total)
        safe = jnp.maximum(cnt, 1.0)
        contrib = jnp.abs(sc / safe - sa / safe) * prop
        contrib = jnp.where(cnt > 0.0, contrib, 0.0)
        out_ref[...] = jnp.sum(contrib, axis=1, keepdims=True)


@functools.partial(jax.jit, static_argnames=("block_rows",))
def _ece_pallas(logits, labels, block_rows):
    n_rows, n_cols = logits.shape
    n_steps = n_rows // block_rows
    body = functools.partial(_ece_body, n_steps=n_steps, n_rows_total=n_rows)
    out = pl.pallas_call(
        body,
        grid=(n_steps,),
        in_specs=[
            pl.BlockSpec((block_rows, n_cols), lambda i: (i, 0)),
            pl.BlockSpec((block_rows, n_cols), lambda i: (i, 0)),
        ],
        out_specs=pl.BlockSpec((1, 1), lambda i: (0, 0)),
        out_shape=jax.ShapeDtypeStruct((1, 1), jnp.float32),
        scratch_shapes=[
            pltpu.VMEM((block_rows, 16), jnp.float32),
            pltpu.VMEM((block_rows, 16), jnp.float32),
            pltpu.VMEM((block_rows, 16), jnp.float32),
        ],
        compiler_params=pltpu.CompilerParams(
            dimension_semantics=("arbitrary",),
        ),
    )(logits, labels)
    return out.reshape(1)


def kernel(logits, labels):
    n_rows = logits.shape[0]
    block_rows = 512 if n_rows % 512 == 0 else n_rows
    return _ece_pallas(logits, labels, block_rows)


# speedup vs baseline: 1.1033x; 1.0571x over previous
import functools

import numpy as np
import jax
import jax.numpy as jnp
from jax.experimental import pallas as pl
from jax.experimental.pallas import tpu as pltpu

_N_BINS = 15


def _ece_body(logits_ref, labels_ref, out_ref, cnt_ref, sconf_ref, sacc_ref,
              *, n_steps, n_rows_total):
    i = pl.program_id(0)

    @pl.when(i == 0)
    def _init():
        cnt_ref[...] = jnp.zeros_like(cnt_ref)
        sconf_ref[...] = jnp.zeros_like(sconf_ref)
        sacc_ref[...] = jnp.zeros_like(sacc_ref)

    x = logits_ref[...]
    y = labels_ref[...]
    r, l = x.shape

    mx = jnp.max(x, axis=1, keepdims=True)
    my = jnp.max(y, axis=1, keepdims=True)
    hit = (x >= mx) & (y >= my)
    acc = jnp.any(hit, axis=1, keepdims=True).astype(jnp.float32)
    conf = jax.nn.sigmoid(mx)

    lane = jax.lax.broadcasted_iota(jnp.int32, (1, 16), 1)
    lane_f = lane.astype(jnp.float32)
    lowers = jnp.where(lane < _N_BINS, lane_f / 15.0, 2.0)
    uppers = jnp.where(lane < _N_BINS, (lane_f + 1.0) / 15.0, -2.0)
    mask = ((conf > lowers) & (conf <= uppers)).astype(jnp.float32)
    cnt_ref[...] += mask
    sconf_ref[...] += mask * conf
    sacc_ref[...] += mask * acc

    @pl.when(i == n_steps - 1)
    def _finish():
        cnt = jnp.sum(cnt_ref[...], axis=0, keepdims=True)
        sc = jnp.sum(sconf_ref[...], axis=0, keepdims=True)
        sa = jnp.sum(sacc_ref[...], axis=0, keepdims=True)
        prop = cnt / jnp.float32(n_rows_total)
        safe = jnp.maximum(cnt, 1.0)
        contrib = jnp.abs(sc / safe - sa / safe) * prop
        contrib = jnp.where(cnt > 0.0, contrib, 0.0)
        out_ref[...] = jnp.sum(contrib, axis=1, keepdims=True)


@functools.partial(jax.jit, static_argnames=("block_rows",))
def _ece_pallas(logits, labels, block_rows):
    n_rows, n_cols = logits.shape
    n_steps = n_rows // block_rows
    body = functools.partial(_ece_body, n_steps=n_steps, n_rows_total=n_rows)
    out = pl.pallas_call(
        body,
        grid=(n_steps,),
        in_specs=[
            pl.BlockSpec((block_rows, n_cols), lambda i: (i, 0)),
            pl.BlockSpec((block_rows, n_cols), lambda i: (i, 0)),
        ],
        out_specs=pl.BlockSpec((1, 1), lambda i: (0, 0)),
        out_shape=jax.ShapeDtypeStruct((1, 1), jnp.float32),
        scratch_shapes=[
            pltpu.VMEM((block_rows, 16), jnp.float32),
            pltpu.VMEM((block_rows, 16), jnp.float32),
            pltpu.VMEM((block_rows, 16), jnp.float32),
        ],
        compiler_params=pltpu.CompilerParams(
            dimension_semantics=("arbitrary",),
        ),
    )(logits, labels)
    return out.reshape(1)


def kernel(logits, labels):
    n_rows = logits.shape[0]
    block_rows = 1024 if n_rows % 1024 == 0 else n_rows
    return _ece_pallas(logits, labels, block_rows)


# speedup vs baseline: 3.5630x; 3.2293x over previous
import functools

import jax
import jax.numpy as jnp
from jax.experimental import pallas as pl
from jax.experimental.pallas import tpu as pltpu

_N_BINS = 15


def _ece_body(logits_ref, labels_ref, out_ref, cnt_ref, sconf_ref, sacc_ref,
              *, n_steps, n_samples):
    i = pl.program_id(0)

    @pl.when(i == 0)
    def _init():
        cnt_ref[...] = jnp.zeros_like(cnt_ref)
        sconf_ref[...] = jnp.zeros_like(sconf_ref)
        sacc_ref[...] = jnp.zeros_like(sacc_ref)

    x = logits_ref[...]
    y = labels_ref[...]

    mx = jnp.max(x, axis=0, keepdims=True)
    my = jnp.max(y, axis=0, keepdims=True)
    hit = (x >= mx) & (y >= my)
    acc = jnp.any(hit, axis=0, keepdims=True).astype(jnp.float32)
    conf = jax.nn.sigmoid(mx)

    b = jax.lax.broadcasted_iota(jnp.int32, (16, 1), 0)
    b_f = b.astype(jnp.float32)
    lowers = jnp.where(b < _N_BINS, b_f / 15.0, 2.0)
    uppers = jnp.where(b < _N_BINS, (b_f + 1.0) / 15.0, -2.0)

    mask = ((conf > lowers) & (conf <= uppers)).astype(jnp.float32)
    cnt_ref[...] += mask
    sconf_ref[...] += mask * conf
    sacc_ref[...] += mask * acc

    @pl.when(i == n_steps - 1)
    def _finish():
        cnt = jnp.sum(cnt_ref[...], axis=1, keepdims=True)
        sc = jnp.sum(sconf_ref[...], axis=1, keepdims=True)
        sa = jnp.sum(sacc_ref[...], axis=1, keepdims=True)
        prop = cnt / jnp.float32(n_samples)
        safe = jnp.maximum(cnt, 1.0)
        contrib = jnp.abs(sc / safe - sa / safe) * prop
        contrib = jnp.where(cnt > 0.0, contrib, 0.0)
        out_ref[...] = jnp.sum(contrib, axis=0, keepdims=True)


@functools.partial(jax.jit, static_argnames=("block_cols",))
def _ece_pallas(logits_t, labels_t, block_cols):
    n_classes, n_samples = logits_t.shape
    n_steps = n_samples // block_cols
    body = functools.partial(_ece_body, n_steps=n_steps, n_samples=n_samples)
    out = pl.pallas_call(
        body,
        grid=(n_steps,),
        in_specs=[
            pl.BlockSpec((n_classes, block_cols), lambda i: (0, i)),
            pl.BlockSpec((n_classes, block_cols), lambda i: (0, i)),
        ],
        out_specs=pl.BlockSpec((1, 1), lambda i: (0, 0)),
        out_shape=jax.ShapeDtypeStruct((1, 1), jnp.float32),
        scratch_shapes=[
            pltpu.VMEM((16, block_cols), jnp.float32),
            pltpu.VMEM((16, block_cols), jnp.float32),
            pltpu.VMEM((16, block_cols), jnp.float32),
        ],
        compiler_params=pltpu.CompilerParams(
            dimension_semantics=("arbitrary",),
        ),
    )(logits_t, labels_t)
    return out.reshape(1)


def kernel(logits, labels):
    n_samples = logits.shape[0]
    block_cols = 512 if n_samples % 512 == 0 else n_samples
    return _ece_pallas(logits.T, labels.T, block_cols)


# speedup vs baseline: 3.9713x; 1.1146x over previous
import functools

import jax
import jax.numpy as jnp
from jax.experimental import pallas as pl
from jax.experimental.pallas import tpu as pltpu

_N_BINS = 15


def _ece_body(logits_ref, labels_ref, out_ref, cnt_ref, sconf_ref, sacc_ref,
              *, n_steps, n_samples):
    i = pl.program_id(0)

    @pl.when(i == 0)
    def _init():
        cnt_ref[...] = jnp.zeros_like(cnt_ref)
        sconf_ref[...] = jnp.zeros_like(sconf_ref)
        sacc_ref[...] = jnp.zeros_like(sacc_ref)

    x = logits_ref[...]
    y = labels_ref[...]

    mx = jnp.max(x, axis=0, keepdims=True)
    my = jnp.max(y, axis=0, keepdims=True)
    hit = (x >= mx) & (y >= my)
    acc = jnp.any(hit, axis=0, keepdims=True).astype(jnp.float32)
    conf = jax.nn.sigmoid(mx)

    b = jax.lax.broadcasted_iota(jnp.int32, (16, 1), 0)
    b_f = b.astype(jnp.float32)
    lowers = jnp.where(b < _N_BINS, b_f / 15.0, 2.0)
    uppers = jnp.where(b < _N_BINS, (b_f + 1.0) / 15.0, -2.0)

    mask = ((conf > lowers) & (conf <= uppers)).astype(jnp.float32)
    cnt_ref[...] += mask
    sconf_ref[...] += mask * conf
    sacc_ref[...] += mask * acc

    @pl.when(i == n_steps - 1)
    def _finish():
        cnt = jnp.sum(cnt_ref[...], axis=1, keepdims=True)
        sc = jnp.sum(sconf_ref[...], axis=1, keepdims=True)
        sa = jnp.sum(sacc_ref[...], axis=1, keepdims=True)
        prop = cnt / jnp.float32(n_samples)
        safe = jnp.maximum(cnt, 1.0)
        contrib = jnp.abs(sc / safe - sa / safe) * prop
        contrib = jnp.where(cnt > 0.0, contrib, 0.0)
        out_ref[...] = jnp.sum(contrib, axis=0, keepdims=True)


@functools.partial(jax.jit, static_argnames=("block_cols",))
def _ece_pallas(logits_t, labels_t, block_cols):
    n_classes, n_samples = logits_t.shape
    n_steps = n_samples // block_cols
    body = functools.partial(_ece_body, n_steps=n_steps, n_samples=n_samples)
    out = pl.pallas_call(
        body,
        grid=(n_steps,),
        in_specs=[
            pl.BlockSpec((n_classes, block_cols), lambda i: (0, i)),
            pl.BlockSpec((n_classes, block_cols), lambda i: (0, i)),
        ],
        out_specs=pl.BlockSpec((1, 1), lambda i: (0, 0)),
        out_shape=jax.ShapeDtypeStruct((1, 1), jnp.float32),
        scratch_shapes=[
            pltpu.VMEM((16, block_cols), jnp.float32),
            pltpu.VMEM((16, block_cols), jnp.float32),
            pltpu.VMEM((16, block_cols), jnp.float32),
        ],
        compiler_params=pltpu.CompilerParams(
            dimension_semantics=("arbitrary",),
        ),
    )(logits_t, labels_t)
    return out.reshape(1)


def kernel(logits, labels):
    n_samples = logits.shape[0]
    block_cols = 1024 if n_samples % 1024 == 0 else n_samples
    return _ece_pallas(logits.T, labels.T, block_cols)


# speedup vs baseline: 4.0893x; 1.0297x over previous
import functools

import jax
import jax.numpy as jnp
from jax.experimental import pallas as pl
from jax.experimental.pallas import tpu as pltpu

_N_BINS = 15


def _ece_body(logits_ref, labels_ref, out_ref, cnt_ref, sconf_ref, sacc_ref,
              *, n_steps, n_samples):
    i = pl.program_id(0)

    @pl.when(i == 0)
    def _init():
        cnt_ref[...] = jnp.zeros_like(cnt_ref)
        sconf_ref[...] = jnp.zeros_like(sconf_ref)
        sacc_ref[...] = jnp.zeros_like(sacc_ref)

    x = logits_ref[...]
    y = labels_ref[...]

    mx = jnp.max(x, axis=0, keepdims=True)
    my = jnp.max(y, axis=0, keepdims=True)
    hit = (x >= mx) & (y >= my)
    acc = jnp.any(hit, axis=0, keepdims=True).astype(jnp.float32)
    conf = jax.nn.sigmoid(mx)

    b = jax.lax.broadcasted_iota(jnp.int32, (16, 1), 0)
    b_f = b.astype(jnp.float32)
    lowers = jnp.where(b < _N_BINS, b_f / 15.0, 2.0)
    uppers = jnp.where(b < _N_BINS, (b_f + 1.0) / 15.0, -2.0)

    mask = ((conf > lowers) & (conf <= uppers)).astype(jnp.float32)
    cnt_ref[...] += mask
    sconf_ref[...] += mask * conf
    sacc_ref[...] += mask * acc

    @pl.when(i == n_steps - 1)
    def _finish():
        cnt = jnp.sum(cnt_ref[...], axis=1, keepdims=True)
        sc = jnp.sum(sconf_ref[...], axis=1, keepdims=True)
        sa = jnp.sum(sacc_ref[...], axis=1, keepdims=True)
        prop = cnt / jnp.float32(n_samples)
        safe = jnp.maximum(cnt, 1.0)
        contrib = jnp.abs(sc / safe - sa / safe) * prop
        contrib = jnp.where(cnt > 0.0, contrib, 0.0)
        out_ref[...] = jnp.sum(contrib, axis=0, keepdims=True)


@functools.partial(jax.jit, static_argnames=("block_cols",))
def _ece_pallas(logits_t, labels_t, block_cols):
    n_classes, n_samples = logits_t.shape
    n_steps = n_samples // block_cols
    body = functools.partial(_ece_body, n_steps=n_steps, n_samples=n_samples)
    out = pl.pallas_call(
        body,
        grid=(n_steps,),
        in_specs=[
            pl.BlockSpec((n_classes, block_cols), lambda i: (0, i)),
            pl.BlockSpec((n_classes, block_cols), lambda i: (0, i)),
        ],
        out_specs=pl.BlockSpec((1, 1), lambda i: (0, 0)),
        out_shape=jax.ShapeDtypeStruct((1, 1), jnp.float32),
        scratch_shapes=[
            pltpu.VMEM((16, block_cols), jnp.float32),
            pltpu.VMEM((16, block_cols), jnp.float32),
            pltpu.VMEM((16, block_cols), jnp.float32),
        ],
        compiler_params=pltpu.CompilerParams(
            dimension_semantics=("arbitrary",),
        ),
    )(logits_t, labels_t)
    return out.reshape(1)


def kernel(logits, labels):
    n_samples = logits.shape[0]
    block_cols = 2048 if n_samples % 2048 == 0 else n_samples
    return _ece_pallas(logits.T, labels.T, block_cols)
